# Initial kernel scaffold; baseline (speedup 1.0000x reference)
import jax
import jax.numpy as jnp
from jax.experimental import pallas as pl


def kernel(y, h_hat_ls, dmrs_ofdm_pos, dmrs_subcarrier_pos):
    raise NotImplementedError("write your pallas kernel here")


# speedup vs baseline: 147.2506x; 147.2506x over previous
import functools

import jax
import jax.numpy as jnp
from jax import lax
from jax.experimental import pallas as pl
from jax.experimental.pallas import tpu as pltpu
from jax.experimental.pallas import tpu_sc as plsc

NTX = 4
NSC = 12
NSYM = 14
NPRB = 273
NRE = NPRB * NSC
NRX = 16
NB = 8
NW = NB * NTX
ROWS_W = NPRB * NSC * NSYM
CHUNK = 2048
NFULL = ROWS_W // CHUNK
TAIL = ROWS_W - NFULL * CHUNK
GSUB = 128


def _focc_body(h_ref, hq_ref):
    row = lax.broadcasted_iota(jnp.int32, (NRE, NRX), 0)
    even = (row % 2) == 0
    for tx in range(NTX):
        x = h_ref[0, :, tx, :]
        up = pltpu.roll(x, NRE - 1, 0)
        dn = pltpu.roll(x, 1, 0)
        partner = jnp.where(even, up, dn)
        hq_ref[0, tx, :, :] = (x + partner) * 0.5


def _idx_pe_body(ofdm_ref, scp_ref, idx_ref, pe_ref):
    b = pl.program_id(0)
    tx = pl.program_id(1)

    j = lax.broadcasted_iota(jnp.int32, (NPRB, NSC * NSYM), 1)
    s = j // NSYM
    y = j % NSYM
    r = y * NSC + s
    sc_r = r // NSYM
    sym_r = r % NSYM
    best = None
    arg = None
    for p in range(12):
        scp = scp_ref[tx, p // 2]
        symp = ofdm_ref[tx, p % 2]
        d = jnp.abs(sc_r - scp) + jnp.abs(sym_r - symp)
        if p == 0:
            best = d
            arg = jnp.zeros_like(d)
        else:
            take = d < best
            best = jnp.where(take, d, best)
            arg = jnp.where(take, p, arg)

    grp = lax.broadcasted_iota(jnp.int32, (NPRB, NSC * NSYM), 0)
    g = grp * NSC + arg
    pg = g // (2 * NPRB)
    dg = (g // NPRB) % 2
    prb = g % NPRB
    mval = dg * (6 * NPRB) + pg * NPRB + prb
    base = (b * NTX + tx) * NRE
    idx_ref[0, 0, :, :] = base + mval

    t_i = lax.broadcasted_iota(jnp.int32, (NRE, NSYM), 0)
    a2 = lax.broadcasted_iota(jnp.int32, (NRE, NSYM), 1)
    b2 = t_i % NSC
    r2 = a2 * NSC + b2
    sc2 = r2 // NSYM
    sym2 = r2 % NSYM
    m0 = None
    m1 = None
    for p in range(12):
        d0 = jnp.abs(sc2 - scp_ref[tx, p // 2])
        d1 = jnp.abs(sym2 - ofdm_ref[tx, p % 2])
        m0 = d0 if m0 is None else jnp.minimum(m0, d0)
        m1 = d1 if m1 is None else jnp.minimum(m1, d1)

    def norm(m):
        mf = m.astype(jnp.float32)
        mean = jnp.sum(mf) / float(NRE * NSYM)
        ctr = mf - mean
        var = jnp.sum(ctr * ctr) / float(NPRB) / float(NSC * NSYM - 1)
        std = jnp.sqrt(var)
        return jnp.where(std > 0.0, ctr / std, ctr)

    pe_ref[0, 0, :, :] = norm(m1)
    pe_ref[0, 1, :, :] = norm(m0)


def _gather_body(hq_ref, idx_ref, out_ref, idx_v, rows0, rows1,
                 gs0, gs1, ss0, ss1, isem):
    cid = lax.axis_index("c")
    sid = lax.axis_index("s")
    w = sid * 2 + cid
    row0 = w * ROWS_W

    pltpu.async_copy(idx_ref.at[pl.ds(row0, ROWS_W)], idx_v, isem).wait()

    def gather_chunk(t, buf, gsem):
        cps = []
        for gsub in range(CHUNK // GSUB):
            off = t * CHUNK + gsub * GSUB
            cps.append(pltpu.async_copy(
                hq_ref.at[idx_v.at[pl.ds(off, GSUB)]],
                buf.at[pl.ds(gsub * GSUB, GSUB)],
                gsem))
        for cp in cps:
            cp.wait()

    def start_store(t, buf, ssem):
        pltpu.async_copy(buf, out_ref.at[pl.ds(row0 + t * CHUNK, CHUNK)], ssem)

    def drain_store(buf, ssem):
        pltpu.make_async_copy(buf, out_ref.at[pl.ds(row0, CHUNK)], ssem).wait()

    def pair(i, _):
        t = i * 2

        @pl.when(t >= 2)
        def _():
            drain_store(rows0, ss0)

        gather_chunk(t, rows0, gs0)
        start_store(t, rows0, ss0)

        @pl.when(t >= 2)
        def _():
            drain_store(rows1, ss1)

        gather_chunk(t + 1, rows1, gs1)
        start_store(t + 1, rows1, ss1)
        return 0

    lax.fori_loop(0, NFULL // 2, pair, 0)

    drain_store(rows0, ss0)
    toff = NFULL * CHUNK
    cps = []
    for gsub in range(TAIL // GSUB):
        cps.append(pltpu.async_copy(
            hq_ref.at[idx_v.at[pl.ds(toff + gsub * GSUB, GSUB)]],
            rows0.at[pl.ds(gsub * GSUB, GSUB)],
            gs0))
    rem = TAIL % GSUB
    cps.append(pltpu.async_copy(
        hq_ref.at[idx_v.at[pl.ds(toff + (TAIL // GSUB) * GSUB, rem)]],
        rows0.at[pl.ds((TAIL // GSUB) * GSUB, rem)],
        gs0))
    for cp in cps:
        cp.wait()
    st = pltpu.async_copy(rows0.at[pl.ds(0, TAIL)],
                          out_ref.at[pl.ds(row0 + toff, TAIL)], ss0)
    st.wait()
    drain_store(rows1, ss1)


def _focc(h_hat_ls, interpret=False):
    return pl.pallas_call(
        _focc_body,
        grid=(NB,),
        in_specs=[pl.BlockSpec((1, NRE, NTX, NRX), lambda b: (b, 0, 0, 0))],
        out_specs=pl.BlockSpec((1, NTX, NRE, NRX), lambda b: (b, 0, 0, 0)),
        out_shape=jax.ShapeDtypeStruct((NB, NTX, NRE, NRX), jnp.float32),
        interpret=interpret,
    )(h_hat_ls)


def _idx_pe(ofdm, scp, interpret=False):
    return pl.pallas_call(
        _idx_pe_body,
        grid=(NB, NTX),
        in_specs=[
            pl.BlockSpec(memory_space=pltpu.SMEM),
            pl.BlockSpec(memory_space=pltpu.SMEM),
        ],
        out_specs=[
            pl.BlockSpec((1, 1, NPRB, NSC * NSYM), lambda b, tx: (b, tx, 0, 0)),
            pl.BlockSpec((1, 2, NRE, NSYM), lambda b, tx: (tx, 0, 0, 0)),
        ],
        out_shape=[
            jax.ShapeDtypeStruct((NB, NTX, NPRB, NSC * NSYM), jnp.int32),
            jax.ShapeDtypeStruct((NTX, 2, NRE, NSYM), jnp.float32),
        ],
        interpret=interpret,
    )(ofdm, scp)


def _gather(hq_flat, idx_flat):
    mesh = plsc.VectorSubcoreMesh(
        core_axis_name="c", subcore_axis_name="s",
        num_cores=2, num_subcores=16)
    run = functools.partial(
        pl.kernel,
        out_type=jax.ShapeDtypeStruct((NW * ROWS_W, NRX), jnp.float32),
        mesh=mesh,
        scratch_types=[
            pltpu.VMEM((ROWS_W,), jnp.int32),
            pltpu.VMEM((CHUNK, NRX), jnp.float32),
            pltpu.VMEM((CHUNK, NRX), jnp.float32),
            pltpu.SemaphoreType.DMA,
            pltpu.SemaphoreType.DMA,
            pltpu.SemaphoreType.DMA,
            pltpu.SemaphoreType.DMA,
            pltpu.SemaphoreType.DMA,
        ],
        compiler_params=pltpu.CompilerParams(use_tc_tiling_on_sc=False),
    )(_gather_body)
    return run(hq_flat, idx_flat)


def kernel(y, h_hat_ls, dmrs_ofdm_pos, dmrs_subcarrier_pos):
    del y
    hq = _focc(h_hat_ls)
    idx, pe_raw = _idx_pe(dmrs_ofdm_pos.astype(jnp.int32),
                          dmrs_subcarrier_pos.astype(jnp.int32))
    out_flat = _gather(hq.reshape(NW * NRE, NRX), idx.reshape(NW * ROWS_W))
    h_hat = out_flat.reshape(NB, NTX, NRE, NSYM, NRX)
    pe = jnp.transpose(pe_raw, (0, 2, 3, 1))
    return (h_hat, pe)
